# Initial kernel scaffold; baseline (speedup 1.0000x reference)
import jax
import jax.numpy as jnp
from jax.experimental import pallas as pl


def kernel(x, edge_index, W1, b1, W_mu, b_mu, W_ls, b_ls):
    raise NotImplementedError("write your pallas kernel here")


# speedup vs baseline: 30.9846x; 30.9846x over previous
import functools

import jax
import jax.numpy as jnp
from jax import lax
from jax.experimental import pallas as pl
from jax.experimental.pallas import tpu as pltpu
from jax.experimental.pallas import tpu_sc as plsc

N_NODES = 10000
N_EDGES = 320000
D_IN = 128
D_HID = 32

NC = 2
NS = 16
NW = NC * NS
CH = 128
K_CH = -(-N_EDGES // (NW * CH))
E_PAD = NW * K_CH * CH
ROW_BLK = 1024
N_PAD = -(-N_NODES // ROW_BLK) * ROW_BLK
NR = N_PAD // NS

_mesh = plsc.VectorSubcoreMesh(
    core_axis_name="c", subcore_axis_name="s", num_cores=NC, num_subcores=NS)
_sc_params = pltpu.CompilerParams(use_tc_tiling_on_sc=False)


@functools.partial(
    pl.kernel,
    out_type=jax.ShapeDtypeStruct((NC, N_PAD, 8), jnp.float32),
    mesh=_mesh,
    scratch_types=[
        pltpu.VMEM((K_CH, CH), jnp.int32),
        pltpu.VMEM((CH, 8), jnp.float32),
        pltpu.VMEM_SHARED((N_PAD, 8), jnp.float32),
    ],
    compiler_params=_sc_params,
)
def _deg_kernel(dst_hbm, zeros8_hbm, ones8_hbm, out_hbm, dst_v, ones_v, acc_sh):
    c = lax.axis_index("c")
    s = lax.axis_index("s")
    w = c * NS + s
    r0 = s * NR
    pltpu.sync_copy(zeros8_hbm.at[pl.ds(r0, NR)], acc_sh.at[pl.ds(r0, NR)])
    pltpu.sync_copy(dst_hbm.at[w], dst_v)
    pltpu.sync_copy(ones8_hbm, ones_v)
    plsc.subcore_barrier()

    def body(j, carry):
        pltpu.sync_copy(ones_v, acc_sh.at[dst_v.at[j]], add=True)
        return carry

    lax.fori_loop(0, K_CH, body, 0)
    plsc.subcore_barrier()
    pltpu.sync_copy(acc_sh.at[pl.ds(r0, NR)], out_hbm.at[c, pl.ds(r0, NR)])


@functools.partial(
    pl.kernel,
    out_type=jax.ShapeDtypeStruct((NC, N_PAD, D_HID), jnp.float32),
    mesh=_mesh,
    scratch_types=[
        pltpu.VMEM((K_CH, CH), jnp.int32),
        pltpu.VMEM((K_CH, CH), jnp.int32),
        pltpu.VMEM((2, CH, D_HID), jnp.float32),
        pltpu.VMEM_SHARED((N_PAD, D_HID), jnp.float32),
        pltpu.SemaphoreType.DMA,
        pltpu.SemaphoreType.DMA,
    ],
    compiler_params=_sc_params,
)
def _spmm_kernel(src_hbm, dst_hbm, g_hbm, zeros_hbm, out_hbm,
                 src_v, dst_v, rows_v, acc_sh, sem0, sem1):
    c = lax.axis_index("c")
    s = lax.axis_index("s")
    w = c * NS + s
    r0 = s * NR
    pltpu.sync_copy(zeros_hbm.at[pl.ds(r0, NR)], acc_sh.at[pl.ds(r0, NR)])
    pltpu.sync_copy(src_hbm.at[w], src_v)
    pltpu.sync_copy(dst_hbm.at[w], dst_v)
    plsc.subcore_barrier()

    del sem1

    def seq_body(j, carry):
        pltpu.async_copy(g_hbm.at[src_v.at[j]], rows_v.at[0], sem0).wait()
        pltpu.sync_copy(rows_v.at[0], acc_sh.at[dst_v.at[j]], add=True)
        return carry

    lax.fori_loop(0, K_CH, seq_body, 0)
    plsc.subcore_barrier()
    pltpu.sync_copy(acc_sh.at[pl.ds(r0, NR)], out_hbm.at[c, pl.ds(r0, NR)])


def _tc1_body(degp_ref, x_ref, w_ref, g_ref, dinv_ref):
    d = degp_ref[0] + degp_ref[1]
    dinv = lax.rsqrt(d[:, 0:1] + 1.0)
    g_ref[...] = jnp.dot(x_ref[...], w_ref[...],
                         preferred_element_type=jnp.float32) * dinv
    dinv_ref[...] = jnp.broadcast_to(dinv, (ROW_BLK, 8))


def _tc2_body(accp_ref, g1_ref, dinv_ref, b1_ref, w_ref, g2_ref):
    dinv = dinv_ref[:, 0:1]
    h = (accp_ref[0] + accp_ref[1] + g1_ref[...]) * dinv + b1_ref[...]
    h = jnp.maximum(h, 0.0)
    g2_ref[...] = jnp.dot(h, w_ref[...],
                          preferred_element_type=jnp.float32) * dinv


def _tc3_body(accp_ref, g2_ref, dinv_ref, b_ref, out_ref):
    dinv = dinv_ref[:, 0:1]
    out_ref[...] = (accp_ref[0] + accp_ref[1] + g2_ref[...]) * dinv + b_ref[...]


_GRID = (N_PAD // ROW_BLK,)


def _row_spec(minor):
    return pl.BlockSpec((ROW_BLK, minor), lambda i: (i, 0))


def _part_spec(minor):
    return pl.BlockSpec((NC, ROW_BLK, minor), lambda i: (0, i, 0))


def _full_spec(a, b):
    return pl.BlockSpec((a, b), lambda i: (0, 0))


_tc1 = pl.pallas_call(
    _tc1_body,
    grid=_GRID,
    in_specs=[_part_spec(8), _row_spec(D_IN), _full_spec(D_IN, D_HID)],
    out_specs=[_row_spec(D_HID), _row_spec(8)],
    out_shape=[jax.ShapeDtypeStruct((N_PAD, D_HID), jnp.float32),
               jax.ShapeDtypeStruct((N_PAD, 8), jnp.float32)],
)

_tc2 = pl.pallas_call(
    _tc2_body,
    grid=_GRID,
    in_specs=[_part_spec(D_HID), _row_spec(D_HID), _row_spec(8),
              _full_spec(1, D_HID), _full_spec(D_HID, D_HID)],
    out_specs=_row_spec(D_HID),
    out_shape=jax.ShapeDtypeStruct((N_PAD, D_HID), jnp.float32),
)

_tc3 = pl.pallas_call(
    _tc3_body,
    grid=_GRID,
    in_specs=[_part_spec(D_HID), _row_spec(D_HID), _row_spec(8),
              _full_spec(1, D_HID)],
    out_specs=_row_spec(D_HID),
    out_shape=jax.ShapeDtypeStruct((N_PAD, D_HID), jnp.float32),
)


def kernel(x, edge_index, W1, b1, W_mu, b_mu, W_ls, b_ls):
    src = edge_index[0].astype(jnp.int32)
    dst = edge_index[1].astype(jnp.int32)
    pad = E_PAD - N_EDGES
    srcp = jnp.concatenate(
        [src, jnp.full((pad,), N_NODES, jnp.int32)]).reshape(NW, K_CH, CH)
    dstp = jnp.concatenate(
        [dst, jnp.full((pad,), N_NODES, jnp.int32)]).reshape(NW, K_CH, CH)
    x_pad = jnp.pad(x, ((0, N_PAD - N_NODES), (0, 0)))
    zeros32 = jnp.zeros((N_PAD, D_HID), jnp.float32)
    zeros8 = jnp.zeros((N_PAD, 8), jnp.float32)
    ones8 = jnp.ones((CH, 8), jnp.float32)
    Wcat = jnp.concatenate([W_mu, W_ls], axis=1)
    bcat = jnp.concatenate([b_mu, b_ls]).reshape(1, D_HID)
    b1r = b1.reshape(1, D_HID)

    degp = _deg_kernel(dstp, zeros8, ones8)
    g1, dinv8 = _tc1(degp, x_pad, W1)
    acc1p = _spmm_kernel(srcp, dstp, g1, zeros32)
    g2 = _tc2(acc1p, g1, dinv8, b1r, Wcat)
    acc2p = _spmm_kernel(srcp, dstp, g2, zeros32)
    f = _tc3(acc2p, g2, dinv8, bcat)
    return (f[:N_NODES, :16], f[:N_NODES, 16:])


# speedup vs baseline: 32.9542x; 1.0636x over previous
import functools

import jax
import jax.numpy as jnp
from jax import lax
from jax.experimental import pallas as pl
from jax.experimental.pallas import tpu as pltpu
from jax.experimental.pallas import tpu_sc as plsc

N_NODES = 10000
N_EDGES = 320000
D_IN = 128
D_HID = 32

NC = 2
NS = 16
NW = NC * NS
CH = 128
NB = 8
K_CH = -(-N_EDGES // (NW * CH * NB)) * NB
E_PAD = NW * K_CH * CH
ROW_BLK = 1024
N_PAD = -(-N_NODES // ROW_BLK) * ROW_BLK
NR = N_PAD // NS

_mesh = plsc.VectorSubcoreMesh(
    core_axis_name="c", subcore_axis_name="s", num_cores=NC, num_subcores=NS)
_sc_params = pltpu.CompilerParams(use_tc_tiling_on_sc=False)


@functools.partial(
    pl.kernel,
    out_type=jax.ShapeDtypeStruct((NC, N_PAD, 8), jnp.float32),
    mesh=_mesh,
    scratch_types=[
        pltpu.VMEM((K_CH, CH), jnp.int32),
        pltpu.VMEM((CH, 8), jnp.float32),
        pltpu.VMEM_SHARED((N_PAD, 8), jnp.float32),
    ],
    compiler_params=_sc_params,
)
def _deg_kernel(dst_hbm, zeros8_hbm, ones8_hbm, out_hbm, dst_v, ones_v, acc_sh):
    c = lax.axis_index("c")
    s = lax.axis_index("s")
    w = c * NS + s
    r0 = s * NR
    pltpu.sync_copy(zeros8_hbm.at[pl.ds(r0, NR)], acc_sh.at[pl.ds(r0, NR)])
    pltpu.sync_copy(dst_hbm.at[w], dst_v)
    pltpu.sync_copy(ones8_hbm, ones_v)
    plsc.subcore_barrier()

    def body(j, carry):
        pltpu.sync_copy(ones_v, acc_sh.at[dst_v.at[j]], add=True)
        return carry

    lax.fori_loop(0, K_CH, body, 0)
    plsc.subcore_barrier()
    pltpu.sync_copy(acc_sh.at[pl.ds(r0, NR)], out_hbm.at[c, pl.ds(r0, NR)])


@functools.partial(
    pl.kernel,
    out_type=jax.ShapeDtypeStruct((NC, N_PAD, D_HID), jnp.float32),
    mesh=_mesh,
    scratch_types=[
        pltpu.VMEM((K_CH, CH), jnp.int32),
        pltpu.VMEM((K_CH, CH), jnp.int32),
        pltpu.VMEM((NB, CH, D_HID), jnp.float32),
        pltpu.VMEM_SHARED((N_PAD, D_HID), jnp.float32),
    ] + [pltpu.SemaphoreType.DMA] * NB,
    compiler_params=_sc_params,
)
def _spmm_kernel(src_hbm, dst_hbm, g_hbm, zeros_hbm, out_hbm,
                 src_v, dst_v, rows_v, acc_sh, *sems):
    c = lax.axis_index("c")
    s = lax.axis_index("s")
    w = c * NS + s
    r0 = s * NR
    pltpu.sync_copy(zeros_hbm.at[pl.ds(r0, NR)], acc_sh.at[pl.ds(r0, NR)])
    pltpu.sync_copy(src_hbm.at[w], src_v)
    pltpu.sync_copy(dst_hbm.at[w], dst_v)
    plsc.subcore_barrier()

    for b in range(NB):
        pltpu.async_copy(g_hbm.at[src_v.at[b]], rows_v.at[b], sems[b])

    def group_body(it, carry):
        j0 = it * NB
        for b in range(NB):
            j = j0 + b
            pltpu.make_async_copy(
                g_hbm.at[src_v.at[j]], rows_v.at[b], sems[b]).wait()
            pltpu.sync_copy(rows_v.at[b], acc_sh.at[dst_v.at[j]], add=True)

            @pl.when(j + NB < K_CH)
            def _prefetch():
                pltpu.async_copy(
                    g_hbm.at[src_v.at[j + NB]], rows_v.at[b], sems[b])
        return carry

    lax.fori_loop(0, K_CH // NB, group_body, 0)
    plsc.subcore_barrier()
    pltpu.sync_copy(acc_sh.at[pl.ds(r0, NR)], out_hbm.at[c, pl.ds(r0, NR)])


def _tc1_body(degp_ref, x_ref, w_ref, g_ref, dinv_ref):
    d = degp_ref[0] + degp_ref[1]
    dinv = lax.rsqrt(d[:, 0:1] + 1.0)
    g_ref[...] = jnp.dot(x_ref[...], w_ref[...],
                         preferred_element_type=jnp.float32) * dinv
    dinv_ref[...] = jnp.broadcast_to(dinv, (ROW_BLK, 8))


def _tc2_body(accp_ref, g1_ref, dinv_ref, b1_ref, w_ref, g2_ref):
    dinv = dinv_ref[:, 0:1]
    h = (accp_ref[0] + accp_ref[1] + g1_ref[...]) * dinv + b1_ref[...]
    h = jnp.maximum(h, 0.0)
    g2_ref[...] = jnp.dot(h, w_ref[...],
                          preferred_element_type=jnp.float32) * dinv


def _tc3_body(accp_ref, g2_ref, dinv_ref, b_ref, out_ref):
    dinv = dinv_ref[:, 0:1]
    out_ref[...] = (accp_ref[0] + accp_ref[1] + g2_ref[...]) * dinv + b_ref[...]


_GRID = (N_PAD // ROW_BLK,)


def _row_spec(minor):
    return pl.BlockSpec((ROW_BLK, minor), lambda i: (i, 0))


def _part_spec(minor):
    return pl.BlockSpec((NC, ROW_BLK, minor), lambda i: (0, i, 0))


def _full_spec(a, b):
    return pl.BlockSpec((a, b), lambda i: (0, 0))


_tc1 = pl.pallas_call(
    _tc1_body,
    grid=_GRID,
    in_specs=[_part_spec(8), _row_spec(D_IN), _full_spec(D_IN, D_HID)],
    out_specs=[_row_spec(D_HID), _row_spec(8)],
    out_shape=[jax.ShapeDtypeStruct((N_PAD, D_HID), jnp.float32),
               jax.ShapeDtypeStruct((N_PAD, 8), jnp.float32)],
)

_tc2 = pl.pallas_call(
    _tc2_body,
    grid=_GRID,
    in_specs=[_part_spec(D_HID), _row_spec(D_HID), _row_spec(8),
              _full_spec(1, D_HID), _full_spec(D_HID, D_HID)],
    out_specs=_row_spec(D_HID),
    out_shape=jax.ShapeDtypeStruct((N_PAD, D_HID), jnp.float32),
)

_tc3 = pl.pallas_call(
    _tc3_body,
    grid=_GRID,
    in_specs=[_part_spec(D_HID), _row_spec(D_HID), _row_spec(8),
              _full_spec(1, D_HID)],
    out_specs=_row_spec(D_HID),
    out_shape=jax.ShapeDtypeStruct((N_PAD, D_HID), jnp.float32),
)


def kernel(x, edge_index, W1, b1, W_mu, b_mu, W_ls, b_ls):
    src = edge_index[0].astype(jnp.int32)
    dst = edge_index[1].astype(jnp.int32)
    pad = E_PAD - N_EDGES
    srcp = jnp.concatenate(
        [src, jnp.full((pad,), N_NODES, jnp.int32)]).reshape(NW, K_CH, CH)
    dstp = jnp.concatenate(
        [dst, jnp.full((pad,), N_NODES, jnp.int32)]).reshape(NW, K_CH, CH)
    x_pad = jnp.pad(x, ((0, N_PAD - N_NODES), (0, 0)))
    zeros32 = jnp.zeros((N_PAD, D_HID), jnp.float32)
    zeros8 = jnp.zeros((N_PAD, 8), jnp.float32)
    ones8 = jnp.ones((CH, 8), jnp.float32)
    Wcat = jnp.concatenate([W_mu, W_ls], axis=1)
    bcat = jnp.concatenate([b_mu, b_ls]).reshape(1, D_HID)
    b1r = b1.reshape(1, D_HID)

    degp = _deg_kernel(dstp, zeros8, ones8)
    g1, dinv8 = _tc1(degp, x_pad, W1)
    acc1p = _spmm_kernel(srcp, dstp, g1, zeros32)
    g2 = _tc2(acc1p, g1, dinv8, b1r, Wcat)
    acc2p = _spmm_kernel(srcp, dstp, g2, zeros32)
    f = _tc3(acc2p, g2, dinv8, bcat)
    return (f[:N_NODES, :16], f[:N_NODES, 16:])


# speedup vs baseline: 60.8766x; 1.8473x over previous
import functools

import jax
import jax.numpy as jnp
from jax import lax
from jax.experimental import pallas as pl
from jax.experimental.pallas import tpu as pltpu
from jax.experimental.pallas import tpu_sc as plsc

N_NODES = 10000
N_EDGES = 320000
D_IN = 128
D_HID = 32

NC = 2
NS = 16
NW = NC * NS
CH = 128
NB = 8
K_CH = -(-N_EDGES // (NW * CH * NB)) * NB
E_PAD = NW * K_CH * CH
ROW_BLK = 1024
N_PAD = -(-N_NODES // ROW_BLK) * ROW_BLK
NR = N_PAD // NS

_mesh = plsc.VectorSubcoreMesh(
    core_axis_name="c", subcore_axis_name="s", num_cores=NC, num_subcores=NS)
_sc_params = pltpu.CompilerParams(use_tc_tiling_on_sc=False)


@functools.partial(
    pl.kernel,
    out_type=jax.ShapeDtypeStruct((NC, N_PAD, 8), jnp.float32),
    mesh=_mesh,
    scratch_types=[
        pltpu.VMEM((K_CH, CH), jnp.int32),
        pltpu.VMEM((CH, 8), jnp.float32),
        pltpu.VMEM_SHARED((N_PAD, 8), jnp.float32),
    ],
    compiler_params=_sc_params,
)
def _deg_kernel(dst_hbm, zeros8_hbm, ones8_hbm, out_hbm, dst_v, ones_v, acc_sh):
    c = lax.axis_index("c")
    s = lax.axis_index("s")
    w = c * NS + s
    r0 = s * NR
    pltpu.sync_copy(zeros8_hbm.at[pl.ds(r0, NR)], acc_sh.at[pl.ds(r0, NR)])
    pltpu.sync_copy(dst_hbm.at[w], dst_v)
    pltpu.sync_copy(ones8_hbm, ones_v)
    plsc.subcore_barrier()

    def body(j, carry):
        pltpu.sync_copy(ones_v, acc_sh.at[dst_v.at[j]], add=True)
        return carry

    lax.fori_loop(0, K_CH, body, 0)
    plsc.subcore_barrier()
    pltpu.sync_copy(acc_sh.at[pl.ds(r0, NR)], out_hbm.at[c, pl.ds(r0, NR)])


@functools.partial(
    pl.kernel,
    out_type=jax.ShapeDtypeStruct((NC, N_PAD, D_HID), jnp.float32),
    mesh=_mesh,
    scratch_types=[
        pltpu.VMEM((K_CH, CH), jnp.int32),
        pltpu.VMEM((K_CH, CH), jnp.int32),
        pltpu.VMEM((NB, CH, D_HID), jnp.float32),
        pltpu.VMEM_SHARED((N_PAD, D_HID), jnp.float32),
    ] + [pltpu.SemaphoreType.DMA] * NB,
    compiler_params=_sc_params,
)
def _spmm_kernel(src_hbm, dst_hbm, g_hbm, zeros_hbm, out_hbm,
                 src_v, dst_v, rows_v, acc_sh, *sems):
    c = lax.axis_index("c")
    s = lax.axis_index("s")
    w = c * NS + s
    r0 = s * NR
    pltpu.sync_copy(zeros_hbm.at[pl.ds(r0, NR)], acc_sh.at[pl.ds(r0, NR)])
    pltpu.sync_copy(src_hbm.at[w], src_v)
    pltpu.sync_copy(dst_hbm.at[w], dst_v)
    plsc.subcore_barrier()

    for b in range(NB):
        pltpu.async_copy(g_hbm.at[src_v.at[b]], rows_v.at[b], sems[b])

    def group_body(it, carry):
        j0 = it * NB
        for b in range(NB):
            j = j0 + b
            pltpu.make_async_copy(
                g_hbm.at[src_v.at[j]], rows_v.at[b], sems[b]).wait()
            pltpu.sync_copy(rows_v.at[b], acc_sh.at[dst_v.at[j]], add=True)

            @pl.when(j + NB < K_CH)
            def _prefetch():
                pltpu.async_copy(
                    g_hbm.at[src_v.at[j + NB]], rows_v.at[b], sems[b])
        return carry

    lax.fori_loop(0, K_CH // NB, group_body, 0)
    plsc.subcore_barrier()
    pltpu.sync_copy(acc_sh.at[pl.ds(r0, NR)], out_hbm.at[c, pl.ds(r0, NR)])


def _tc1_body(degp_ref, x_ref, w_ref, g_ref, dinv_ref):
    d = degp_ref[0] + degp_ref[1]
    dinv = lax.rsqrt(d[:, 0:1] + 1.0)
    g_ref[...] = jnp.dot(x_ref[...], w_ref[...],
                         preferred_element_type=jnp.float32) * dinv
    dinv_ref[...] = jnp.broadcast_to(dinv, (ROW_BLK, 8))


def _tc2_body(accp_ref, g1_ref, dinv_ref, b1_ref, w_ref, g2_ref):
    dinv = dinv_ref[:, 0:1]
    h = (accp_ref[0] + accp_ref[1] + g1_ref[...]) * dinv + b1_ref[...]
    h = jnp.maximum(h, 0.0)
    g2_ref[...] = jnp.dot(h, w_ref[...],
                          preferred_element_type=jnp.float32) * dinv


def _tc3_body(accp_ref, g2_ref, dinv_ref, b_ref, out_ref):
    dinv = dinv_ref[:, 0:1]
    out_ref[...] = (accp_ref[0] + accp_ref[1] + g2_ref[...]) * dinv + b_ref[...]


_GRID = (N_PAD // ROW_BLK,)


def _row_spec(minor):
    return pl.BlockSpec((ROW_BLK, minor), lambda i: (i, 0))


def _part_spec(minor):
    return pl.BlockSpec((NC, ROW_BLK, minor), lambda i: (0, i, 0))


def _full_spec(a, b):
    return pl.BlockSpec((a, b), lambda i: (0, 0))


_tc1 = pl.pallas_call(
    _tc1_body,
    grid=_GRID,
    in_specs=[_part_spec(8), _row_spec(D_IN), _full_spec(D_IN, D_HID)],
    out_specs=[_row_spec(D_HID), _row_spec(8)],
    out_shape=[jax.ShapeDtypeStruct((N_PAD, D_HID), jnp.float32),
               jax.ShapeDtypeStruct((N_PAD, 8), jnp.float32)],
)

_tc2 = pl.pallas_call(
    _tc2_body,
    grid=_GRID,
    in_specs=[_part_spec(D_HID), _row_spec(D_HID), _row_spec(8),
              _full_spec(1, D_HID), _full_spec(D_HID, D_HID)],
    out_specs=_row_spec(D_HID),
    out_shape=jax.ShapeDtypeStruct((N_PAD, D_HID), jnp.float32),
)

_tc3 = pl.pallas_call(
    _tc3_body,
    grid=_GRID,
    in_specs=[_part_spec(D_HID), _row_spec(D_HID), _row_spec(8),
              _full_spec(1, D_HID)],
    out_specs=_row_spec(D_HID),
    out_shape=jax.ShapeDtypeStruct((N_PAD, D_HID), jnp.float32),
)


def kernel(x, edge_index, W1, b1, W_mu, b_mu, W_ls, b_ls):
    src = edge_index[0].astype(jnp.int32)
    dst = edge_index[1].astype(jnp.int32)
    pad = E_PAD - N_EDGES
    pad_idx = N_NODES + jnp.arange(pad, dtype=jnp.int32) % (N_PAD - N_NODES)
    srcp = jnp.concatenate([src, pad_idx]).reshape(NW, K_CH, CH)
    dstp = jnp.concatenate([dst, pad_idx]).reshape(NW, K_CH, CH)
    x_pad = jnp.pad(x, ((0, N_PAD - N_NODES), (0, 0)))
    zeros32 = jnp.zeros((N_PAD, D_HID), jnp.float32)
    zeros8 = jnp.zeros((N_PAD, 8), jnp.float32)
    ones8 = jnp.ones((CH, 8), jnp.float32)
    Wcat = jnp.concatenate([W_mu, W_ls], axis=1)
    bcat = jnp.concatenate([b_mu, b_ls]).reshape(1, D_HID)
    b1r = b1.reshape(1, D_HID)

    degp = _deg_kernel(dstp, zeros8, ones8)
    g1, dinv8 = _tc1(degp, x_pad, W1)
    acc1p = _spmm_kernel(srcp, dstp, g1, zeros32)
    g2 = _tc2(acc1p, g1, dinv8, b1r, Wcat)
    acc2p = _spmm_kernel(srcp, dstp, g2, zeros32)
    f = _tc3(acc2p, g2, dinv8, bcat)
    return (f[:N_NODES, :16], f[:N_NODES, 16:])


# speedup vs baseline: 61.0405x; 1.0027x over previous
import functools

import jax
import jax.numpy as jnp
from jax import lax
from jax.experimental import pallas as pl
from jax.experimental.pallas import tpu as pltpu
from jax.experimental.pallas import tpu_sc as plsc

N_NODES = 10000
N_EDGES = 320000
D_IN = 128
D_HID = 32

NC = 2
NS = 16
NW = NC * NS
CH = 128
NB = 8
E_ROWS = N_EDGES // CH
K_MAX = 79
ROW_BLK = 1024
N_PAD = -(-N_NODES // ROW_BLK) * ROW_BLK
NR = N_PAD // NS

_mesh = plsc.VectorSubcoreMesh(
    core_axis_name="c", subcore_axis_name="s", num_cores=NC, num_subcores=NS)
_sc_params = pltpu.CompilerParams(use_tc_tiling_on_sc=False)


def _worker_rows(w):
    wm = jnp.minimum(w, 4)
    base = w * 78 + wm
    nch = 78 + (w < 4).astype(jnp.int32)
    return base, nch


def _load_idx(hbm2d, vref, base, w):
    pltpu.sync_copy(hbm2d.at[pl.ds(base, 78)], vref.at[pl.ds(0, 78)])

    @pl.when(w < 4)
    def _tail():
        pltpu.sync_copy(hbm2d.at[pl.ds(base + 78, 1)], vref.at[pl.ds(78, 1)])


@functools.partial(
    pl.kernel,
    out_type=jax.ShapeDtypeStruct((NC, N_PAD, 8), jnp.float32),
    mesh=_mesh,
    scratch_types=[
        pltpu.VMEM((K_MAX, CH), jnp.int32),
        pltpu.VMEM((CH, 8), jnp.float32),
        pltpu.VMEM_SHARED((N_PAD, 8), jnp.float32),
    ],
    compiler_params=_sc_params,
)
def _deg_kernel(dst_hbm, zeros8_hbm, ones8_hbm, out_hbm, dst_v, ones_v, acc_sh):
    c = lax.axis_index("c")
    s = lax.axis_index("s")
    w = c * NS + s
    r0 = s * NR
    base, nch = _worker_rows(w)
    pltpu.sync_copy(zeros8_hbm.at[pl.ds(r0, NR)], acc_sh.at[pl.ds(r0, NR)])
    _load_idx(dst_hbm, dst_v, base, w)
    pltpu.sync_copy(ones8_hbm, ones_v)
    plsc.subcore_barrier()

    def body(j, carry):
        pltpu.sync_copy(ones_v, acc_sh.at[dst_v.at[j]], add=True)
        return carry

    lax.fori_loop(0, nch, body, 0)
    plsc.subcore_barrier()
    pltpu.sync_copy(acc_sh.at[pl.ds(r0, NR)], out_hbm.at[c, pl.ds(r0, NR)])


@functools.partial(
    pl.kernel,
    out_type=jax.ShapeDtypeStruct((NC, N_PAD, D_HID), jnp.float32),
    mesh=_mesh,
    scratch_types=[
        pltpu.VMEM((K_MAX, CH), jnp.int32),
        pltpu.VMEM((K_MAX, CH), jnp.int32),
        pltpu.VMEM((NB, CH, D_HID), jnp.float32),
        pltpu.VMEM_SHARED((N_PAD, D_HID), jnp.float32),
        pltpu.SemaphoreType.DMA((NB,)),
    ],
    compiler_params=_sc_params,
)
def _spmm_kernel(src_hbm, dst_hbm, g_hbm, zeros_hbm, out_hbm,
                 src_v, dst_v, rows_v, acc_sh, sem):
    c = lax.axis_index("c")
    s = lax.axis_index("s")
    w = c * NS + s
    r0 = s * NR
    base, nch = _worker_rows(w)
    pltpu.sync_copy(zeros_hbm.at[pl.ds(r0, NR)], acc_sh.at[pl.ds(r0, NR)])
    _load_idx(src_hbm, src_v, base, w)
    _load_idx(dst_hbm, dst_v, base, w)
    plsc.subcore_barrier()

    for b in range(NB):
        pltpu.async_copy(g_hbm.at[src_v.at[b]], rows_v.at[b], sem.at[b])

    def body(j, carry):
        slot = lax.rem(j, NB)
        pltpu.make_async_copy(
            g_hbm.at[src_v.at[j]], rows_v.at[slot], sem.at[slot]).wait()
        pltpu.sync_copy(rows_v.at[slot], acc_sh.at[dst_v.at[j]], add=True)

        @pl.when(j + NB < nch)
        def _prefetch():
            pltpu.async_copy(
                g_hbm.at[src_v.at[j + NB]], rows_v.at[slot], sem.at[slot])

        return carry

    lax.fori_loop(0, nch, body, 0)
    plsc.subcore_barrier()
    pltpu.sync_copy(acc_sh.at[pl.ds(r0, NR)], out_hbm.at[c, pl.ds(r0, NR)])


def _mm_body(x_ref, w_ref, o_ref):
    o_ref[...] = jnp.dot(x_ref[...], w_ref[...],
                         preferred_element_type=jnp.float32)


def _scale_body(degp_ref, xw_ref, g_ref, dinv_ref):
    d = degp_ref[0] + degp_ref[1]
    dinv = lax.rsqrt(d[:, 0:1] + 1.0)
    g_ref[...] = xw_ref[...] * dinv
    dinv_ref[...] = jnp.broadcast_to(dinv, (ROW_BLK, 8))


def _tc2_body(accp_ref, g1_ref, dinv_ref, b1_ref, w_ref, g2_ref):
    dinv = dinv_ref[:, 0:1]
    h = (accp_ref[0] + accp_ref[1] + g1_ref[...]) * dinv + b1_ref[...]
    h = jnp.maximum(h, 0.0)
    g2_ref[...] = jnp.dot(h, w_ref[...],
                          preferred_element_type=jnp.float32) * dinv


def _tc3_body(accp_ref, g2_ref, dinv_ref, b_ref, mu_ref, ls_ref):
    dinv = dinv_ref[:, 0:1]
    f = (accp_ref[0] + accp_ref[1] + g2_ref[...]) * dinv + b_ref[...]
    mu_ref[...] = f[:, :16]
    ls_ref[...] = f[:, 16:]


def _row_spec(rows, minor):
    return pl.BlockSpec((rows, minor), lambda i: (i, 0))


def _part_spec(rows, minor):
    return pl.BlockSpec((NC, rows, minor), lambda i: (0, i, 0))


def _full_spec(a, b):
    return pl.BlockSpec((a, b), lambda i: (0, 0))


_tc_mm = pl.pallas_call(
    _mm_body,
    grid=(N_PAD // ROW_BLK,),
    in_specs=[_row_spec(ROW_BLK, D_IN), _full_spec(D_IN, D_HID)],
    out_specs=_row_spec(ROW_BLK, D_HID),
    out_shape=jax.ShapeDtypeStruct((N_PAD, D_HID), jnp.float32),
)

_tc_scale = pl.pallas_call(
    _scale_body,
    grid=(N_PAD // ROW_BLK,),
    in_specs=[_part_spec(ROW_BLK, 8), _row_spec(ROW_BLK, D_HID)],
    out_specs=[_row_spec(ROW_BLK, D_HID), _row_spec(ROW_BLK, 8)],
    out_shape=[jax.ShapeDtypeStruct((N_PAD, D_HID), jnp.float32),
               jax.ShapeDtypeStruct((N_PAD, 8), jnp.float32)],
)

_tc2 = pl.pallas_call(
    _tc2_body,
    grid=(N_PAD // ROW_BLK,),
    in_specs=[_part_spec(ROW_BLK, D_HID), _row_spec(ROW_BLK, D_HID),
              _row_spec(ROW_BLK, 8), _full_spec(1, D_HID),
              _full_spec(D_HID, D_HID)],
    out_specs=_row_spec(ROW_BLK, D_HID),
    out_shape=jax.ShapeDtypeStruct((N_PAD, D_HID), jnp.float32),
)

_OUT_BLK = 1000

_tc3 = pl.pallas_call(
    _tc3_body,
    grid=(N_NODES // _OUT_BLK,),
    in_specs=[_part_spec(_OUT_BLK, D_HID), _row_spec(_OUT_BLK, D_HID),
              _row_spec(_OUT_BLK, 8), _full_spec(1, D_HID)],
    out_specs=[_row_spec(_OUT_BLK, 16), _row_spec(_OUT_BLK, 16)],
    out_shape=[jax.ShapeDtypeStruct((N_NODES, 16), jnp.float32),
               jax.ShapeDtypeStruct((N_NODES, 16), jnp.float32)],
)


def kernel(x, edge_index, W1, b1, W_mu, b_mu, W_ls, b_ls):
    ei32 = edge_index.astype(jnp.int32)
    src2d = ei32[0].reshape(E_ROWS, CH)
    dst2d = ei32[1].reshape(E_ROWS, CH)
    x_pad = jnp.pad(x, ((0, N_PAD - N_NODES), (0, 0)))
    zeros32 = jnp.zeros((N_PAD, D_HID), jnp.float32)
    zeros8 = jnp.zeros((N_PAD, 8), jnp.float32)
    ones8 = jnp.ones((CH, 8), jnp.float32)
    Wcat = jnp.concatenate([W_mu, W_ls], axis=1)
    bcat = jnp.concatenate([b_mu, b_ls]).reshape(1, D_HID)

    degp = _deg_kernel(dst2d, zeros8, ones8)
    xw1 = _tc_mm(x_pad, W1)
    g1, dinv8 = _tc_scale(degp, xw1)
    acc1p = _spmm_kernel(src2d, dst2d, g1, zeros32)
    g2 = _tc2(acc1p, g1, dinv8, b1.reshape(1, D_HID), Wcat)
    acc2p = _spmm_kernel(src2d, dst2d, g2, zeros32)
    return tuple(_tc3(acc2p, g2, dinv8, bcat))


# speedup vs baseline: 61.1321x; 1.0015x over previous
import functools

import jax
import jax.numpy as jnp
from jax import lax
from jax.experimental import pallas as pl
from jax.experimental.pallas import tpu as pltpu
from jax.experimental.pallas import tpu_sc as plsc

N_NODES = 10000
N_EDGES = 320000
D_IN = 128
D_HID = 32

NC = 2
NS = 16
NW = NC * NS
CH = 128
NB = 8
E_ROWS = N_EDGES // CH
K_MAX = 79
ROW_BLK = 1024
N_PAD = -(-N_NODES // ROW_BLK) * ROW_BLK
NR = N_PAD // NS

_mesh = plsc.VectorSubcoreMesh(
    core_axis_name="c", subcore_axis_name="s", num_cores=NC, num_subcores=NS)
_sc_params = pltpu.CompilerParams(use_tc_tiling_on_sc=False)


def _worker_rows(w):
    wm = jnp.minimum(w, 4)
    base = w * 78 + wm
    nch = 78 + (w < 4).astype(jnp.int32)
    return base, nch


def _load_idx(hbm2d, vref, base, w):
    pltpu.sync_copy(hbm2d.at[pl.ds(base, 78)], vref.at[pl.ds(0, 78)])

    @pl.when(w < 4)
    def _tail():
        pltpu.sync_copy(hbm2d.at[pl.ds(base + 78, 1)], vref.at[pl.ds(78, 1)])


@functools.partial(
    pl.kernel,
    out_type=jax.ShapeDtypeStruct((NC, N_PAD, 8), jnp.float32),
    mesh=_mesh,
    scratch_types=[
        pltpu.VMEM((K_MAX, CH), jnp.int32),
        pltpu.VMEM((CH, 8), jnp.float32),
        pltpu.VMEM_SHARED((N_PAD, 8), jnp.float32),
    ],
    compiler_params=_sc_params,
)
def _deg_kernel(dst_hbm, zeros8_hbm, ones8_hbm, out_hbm, dst_v, ones_v, acc_sh):
    c = lax.axis_index("c")
    s = lax.axis_index("s")
    w = c * NS + s
    r0 = s * NR
    base, nch = _worker_rows(w)
    pltpu.sync_copy(zeros8_hbm.at[pl.ds(r0, NR)], acc_sh.at[pl.ds(r0, NR)])
    _load_idx(dst_hbm, dst_v, base, w)
    pltpu.sync_copy(ones8_hbm, ones_v)
    plsc.subcore_barrier()

    def body(j, carry):
        pltpu.sync_copy(ones_v, acc_sh.at[dst_v.at[j]], add=True)
        return carry

    lax.fori_loop(0, nch, body, 0)
    plsc.subcore_barrier()
    pltpu.sync_copy(acc_sh.at[pl.ds(r0, NR)], out_hbm.at[c, pl.ds(r0, NR)])


@functools.partial(
    pl.kernel,
    out_type=jax.ShapeDtypeStruct((NC, N_PAD, D_HID), jnp.float32),
    mesh=_mesh,
    scratch_types=[
        pltpu.VMEM((K_MAX, CH), jnp.int32),
        pltpu.VMEM((K_MAX, CH), jnp.int32),
        pltpu.VMEM((NB, CH, D_HID), jnp.float32),
        pltpu.VMEM_SHARED((N_PAD, D_HID), jnp.float32),
        pltpu.SemaphoreType.DMA((NB,)),
    ],
    compiler_params=_sc_params,
)
def _spmm_kernel(src_hbm, dst_hbm, g_hbm, zeros_hbm, out_hbm,
                 src_v, dst_v, rows_v, acc_sh, sem_g):
    c = lax.axis_index("c")
    s = lax.axis_index("s")
    w = c * NS + s
    r0 = s * NR
    base, nch = _worker_rows(w)
    pltpu.sync_copy(zeros_hbm.at[pl.ds(r0, NR)], acc_sh.at[pl.ds(r0, NR)])
    _load_idx(src_hbm, src_v, base, w)
    _load_idx(dst_hbm, dst_v, base, w)
    plsc.subcore_barrier()

    for b in range(NB):
        pltpu.async_copy(g_hbm.at[src_v.at[b]], rows_v.at[b], sem_g.at[b])

    def body(j, carry):
        slot = lax.rem(j, NB)
        pltpu.make_async_copy(
            g_hbm.at[src_v.at[j]], rows_v.at[slot], sem_g.at[slot]).wait()
        pltpu.sync_copy(rows_v.at[slot], acc_sh.at[dst_v.at[j]], add=True)

        @pl.when(j + NB < nch)
        def _prefetch():
            pltpu.async_copy(
                g_hbm.at[src_v.at[j + NB]], rows_v.at[slot], sem_g.at[slot])

        return carry

    lax.fori_loop(0, nch, body, 0)
    plsc.subcore_barrier()
    pltpu.sync_copy(acc_sh.at[pl.ds(r0, NR)], out_hbm.at[c, pl.ds(r0, NR)])


def _mm_body(x_ref, w_ref, o_ref):
    o_ref[...] = jnp.dot(x_ref[...], w_ref[...],
                         preferred_element_type=jnp.float32)


def _scale_body(degp_ref, xw_ref, g_ref, dinv_ref):
    d = degp_ref[0] + degp_ref[1]
    dinv = lax.rsqrt(d[:, 0:1] + 1.0)
    g_ref[...] = xw_ref[...] * dinv
    dinv_ref[...] = jnp.broadcast_to(dinv, (ROW_BLK, 8))


def _tc2_body(accp_ref, g1_ref, dinv_ref, b1_ref, w_ref, g2_ref):
    dinv = dinv_ref[:, 0:1]
    h = (accp_ref[0] + accp_ref[1] + g1_ref[...]) * dinv + b1_ref[...]
    h = jnp.maximum(h, 0.0)
    g2_ref[...] = jnp.dot(h, w_ref[...],
                          preferred_element_type=jnp.float32) * dinv


def _tc3_body(accp_ref, g2_ref, dinv_ref, b_ref, mu_ref, ls_ref):
    dinv = dinv_ref[:, 0:1]
    f = (accp_ref[0] + accp_ref[1] + g2_ref[...]) * dinv + b_ref[...]
    mu_ref[...] = f[:, :16]
    ls_ref[...] = f[:, 16:]


def _row_spec(rows, minor):
    return pl.BlockSpec((rows, minor), lambda i: (i, 0))


def _part_spec(rows, minor):
    return pl.BlockSpec((NC, rows, minor), lambda i: (0, i, 0))


def _full_spec(a, b):
    return pl.BlockSpec((a, b), lambda i: (0, 0))


_tc_mm = pl.pallas_call(
    _mm_body,
    grid=(N_PAD // ROW_BLK,),
    in_specs=[_row_spec(ROW_BLK, D_IN), _full_spec(D_IN, D_HID)],
    out_specs=_row_spec(ROW_BLK, D_HID),
    out_shape=jax.ShapeDtypeStruct((N_PAD, D_HID), jnp.float32),
)

_tc_scale = pl.pallas_call(
    _scale_body,
    grid=(N_PAD // ROW_BLK,),
    in_specs=[_part_spec(ROW_BLK, 8), _row_spec(ROW_BLK, D_HID)],
    out_specs=[_row_spec(ROW_BLK, D_HID), _row_spec(ROW_BLK, 8)],
    out_shape=[jax.ShapeDtypeStruct((N_PAD, D_HID), jnp.float32),
               jax.ShapeDtypeStruct((N_PAD, 8), jnp.float32)],
)

_tc2 = pl.pallas_call(
    _tc2_body,
    grid=(N_PAD // ROW_BLK,),
    in_specs=[_part_spec(ROW_BLK, D_HID), _row_spec(ROW_BLK, D_HID),
              _row_spec(ROW_BLK, 8), _full_spec(1, D_HID),
              _full_spec(D_HID, D_HID)],
    out_specs=_row_spec(ROW_BLK, D_HID),
    out_shape=jax.ShapeDtypeStruct((N_PAD, D_HID), jnp.float32),
)

_OUT_BLK = 1000

_tc3 = pl.pallas_call(
    _tc3_body,
    grid=(N_NODES // _OUT_BLK,),
    in_specs=[_part_spec(_OUT_BLK, D_HID), _row_spec(_OUT_BLK, D_HID),
              _row_spec(_OUT_BLK, 8), _full_spec(1, D_HID)],
    out_specs=[_row_spec(_OUT_BLK, 16), _row_spec(_OUT_BLK, 16)],
    out_shape=[jax.ShapeDtypeStruct((N_NODES, 16), jnp.float32),
               jax.ShapeDtypeStruct((N_NODES, 16), jnp.float32)],
)


def kernel(x, edge_index, W1, b1, W_mu, b_mu, W_ls, b_ls):
    ei32 = edge_index.astype(jnp.int32)
    src2d = ei32[0].reshape(E_ROWS, CH)
    dst2d = ei32[1].reshape(E_ROWS, CH)
    x_pad = jnp.pad(x, ((0, N_PAD - N_NODES), (0, 0)))
    zeros32 = jnp.zeros((N_PAD, D_HID), jnp.float32)
    zeros8 = jnp.zeros((N_PAD, 8), jnp.float32)
    ones8 = jnp.ones((CH, 8), jnp.float32)
    Wcat = jnp.concatenate([W_mu, W_ls], axis=1)
    bcat = jnp.concatenate([b_mu, b_ls]).reshape(1, D_HID)

    degp = _deg_kernel(dst2d, zeros8, ones8)
    xw1 = _tc_mm(x_pad, W1)
    g1, dinv8 = _tc_scale(degp, xw1)
    acc1p = _spmm_kernel(src2d, dst2d, g1, zeros32)
    g2 = _tc2(acc1p, g1, dinv8, b1.reshape(1, D_HID), Wcat)
    acc2p = _spmm_kernel(src2d, dst2d, g2, zeros32)
    return tuple(_tc3(acc2p, g2, dinv8, bcat))


# speedup vs baseline: 61.6377x; 1.0083x over previous
import functools

import jax
import jax.numpy as jnp
from jax import lax
from jax.experimental import pallas as pl
from jax.experimental.pallas import tpu as pltpu
from jax.experimental.pallas import tpu_sc as plsc

N_NODES = 10000
N_EDGES = 320000
D_IN = 128
D_HID = 32

NC = 2
NS = 16
NW = NC * NS
CH = 256
NB = 8
E_ROWS = N_EDGES // CH
K_MAX = 40
ROW_BLK = 1024
N_PAD = -(-N_NODES // ROW_BLK) * ROW_BLK
NR = N_PAD // NS

_mesh = plsc.VectorSubcoreMesh(
    core_axis_name="c", subcore_axis_name="s", num_cores=NC, num_subcores=NS)
_sc_params = pltpu.CompilerParams(use_tc_tiling_on_sc=False)


def _worker_rows(w):
    wm = jnp.minimum(w, 2)
    base = w * 39 + wm
    nch = 39 + (w < 2).astype(jnp.int32)
    return base, nch


def _load_idx(hbm2d, vref, base, w):
    pltpu.sync_copy(hbm2d.at[pl.ds(base, 39)], vref.at[pl.ds(0, 39)])

    @pl.when(w < 2)
    def _tail():
        pltpu.sync_copy(hbm2d.at[pl.ds(base + 39, 1)], vref.at[pl.ds(39, 1)])


@functools.partial(
    pl.kernel,
    out_type=jax.ShapeDtypeStruct((NC, N_PAD, 8), jnp.float32),
    mesh=_mesh,
    scratch_types=[
        pltpu.VMEM((K_MAX, CH), jnp.int32),
        pltpu.VMEM((CH, 8), jnp.float32),
        pltpu.VMEM_SHARED((N_PAD, 8), jnp.float32),
    ],
    compiler_params=_sc_params,
)
def _deg_kernel(dst_hbm, zeros8_hbm, ones8_hbm, out_hbm, dst_v, ones_v, acc_sh):
    c = lax.axis_index("c")
    s = lax.axis_index("s")
    w = c * NS + s
    r0 = s * NR
    base, nch = _worker_rows(w)
    pltpu.sync_copy(zeros8_hbm.at[pl.ds(r0, NR)], acc_sh.at[pl.ds(r0, NR)])
    _load_idx(dst_hbm, dst_v, base, w)
    pltpu.sync_copy(ones8_hbm, ones_v)
    plsc.subcore_barrier()

    def body(j, carry):
        pltpu.sync_copy(ones_v, acc_sh.at[dst_v.at[j]], add=True)
        return carry

    lax.fori_loop(0, nch, body, 0)
    plsc.subcore_barrier()
    pltpu.sync_copy(acc_sh.at[pl.ds(r0, NR)], out_hbm.at[c, pl.ds(r0, NR)])


@functools.partial(
    pl.kernel,
    out_type=jax.ShapeDtypeStruct((NC, N_PAD, D_HID), jnp.float32),
    mesh=_mesh,
    scratch_types=[
        pltpu.VMEM((K_MAX, CH), jnp.int32),
        pltpu.VMEM((K_MAX, CH), jnp.int32),
        pltpu.VMEM((NB, CH, D_HID), jnp.float32),
        pltpu.VMEM_SHARED((N_PAD, D_HID), jnp.float32),
        pltpu.SemaphoreType.DMA((NB,)),
    ],
    compiler_params=_sc_params,
)
def _spmm_kernel(src_hbm, dst_hbm, g_hbm, zeros_hbm, out_hbm,
                 src_v, dst_v, rows_v, acc_sh, sem_g):
    c = lax.axis_index("c")
    s = lax.axis_index("s")
    w = c * NS + s
    r0 = s * NR
    base, nch = _worker_rows(w)
    pltpu.sync_copy(zeros_hbm.at[pl.ds(r0, NR)], acc_sh.at[pl.ds(r0, NR)])
    _load_idx(src_hbm, src_v, base, w)
    _load_idx(dst_hbm, dst_v, base, w)
    plsc.subcore_barrier()

    for b in range(NB):
        pltpu.async_copy(g_hbm.at[src_v.at[b]], rows_v.at[b], sem_g.at[b])

    def body(j, carry):
        slot = lax.rem(j, NB)
        pltpu.make_async_copy(
            g_hbm.at[src_v.at[j]], rows_v.at[slot], sem_g.at[slot]).wait()
        pltpu.sync_copy(rows_v.at[slot], acc_sh.at[dst_v.at[j]], add=True)

        @pl.when(j + NB < nch)
        def _prefetch():
            pltpu.async_copy(
                g_hbm.at[src_v.at[j + NB]], rows_v.at[slot], sem_g.at[slot])

        return carry

    lax.fori_loop(0, nch, body, 0)
    plsc.subcore_barrier()
    pltpu.sync_copy(acc_sh.at[pl.ds(r0, NR)], out_hbm.at[c, pl.ds(r0, NR)])


def _mm_body(x_ref, w_ref, o_ref):
    o_ref[...] = jnp.dot(x_ref[...], w_ref[...],
                         preferred_element_type=jnp.float32)


def _scale_body(degp_ref, xw_ref, g_ref, dinv_ref):
    d = degp_ref[0] + degp_ref[1]
    dinv = lax.rsqrt(d[:, 0:1] + 1.0)
    g_ref[...] = xw_ref[...] * dinv
    dinv_ref[...] = jnp.broadcast_to(dinv, (ROW_BLK, 8))


def _tc2_body(accp_ref, g1_ref, dinv_ref, b1_ref, w_ref, g2_ref):
    dinv = dinv_ref[:, 0:1]
    h = (accp_ref[0] + accp_ref[1] + g1_ref[...]) * dinv + b1_ref[...]
    h = jnp.maximum(h, 0.0)
    g2_ref[...] = jnp.dot(h, w_ref[...],
                          preferred_element_type=jnp.float32) * dinv


def _tc3_body(accp_ref, g2_ref, dinv_ref, b_ref, mu_ref, ls_ref):
    dinv = dinv_ref[:, 0:1]
    f = (accp_ref[0] + accp_ref[1] + g2_ref[...]) * dinv + b_ref[...]
    mu_ref[...] = f[:, :16]
    ls_ref[...] = f[:, 16:]


def _row_spec(rows, minor):
    return pl.BlockSpec((rows, minor), lambda i: (i, 0))


def _part_spec(rows, minor):
    return pl.BlockSpec((NC, rows, minor), lambda i: (0, i, 0))


def _full_spec(a, b):
    return pl.BlockSpec((a, b), lambda i: (0, 0))


_tc_mm = pl.pallas_call(
    _mm_body,
    grid=(N_PAD // ROW_BLK,),
    in_specs=[_row_spec(ROW_BLK, D_IN), _full_spec(D_IN, D_HID)],
    out_specs=_row_spec(ROW_BLK, D_HID),
    out_shape=jax.ShapeDtypeStruct((N_PAD, D_HID), jnp.float32),
)

_tc_scale = pl.pallas_call(
    _scale_body,
    grid=(N_PAD // ROW_BLK,),
    in_specs=[_part_spec(ROW_BLK, 8), _row_spec(ROW_BLK, D_HID)],
    out_specs=[_row_spec(ROW_BLK, D_HID), _row_spec(ROW_BLK, 8)],
    out_shape=[jax.ShapeDtypeStruct((N_PAD, D_HID), jnp.float32),
               jax.ShapeDtypeStruct((N_PAD, 8), jnp.float32)],
)

_tc2 = pl.pallas_call(
    _tc2_body,
    grid=(N_PAD // ROW_BLK,),
    in_specs=[_part_spec(ROW_BLK, D_HID), _row_spec(ROW_BLK, D_HID),
              _row_spec(ROW_BLK, 8), _full_spec(1, D_HID),
              _full_spec(D_HID, D_HID)],
    out_specs=_row_spec(ROW_BLK, D_HID),
    out_shape=jax.ShapeDtypeStruct((N_PAD, D_HID), jnp.float32),
)

_OUT_BLK = 1000

_tc3 = pl.pallas_call(
    _tc3_body,
    grid=(N_NODES // _OUT_BLK,),
    in_specs=[_part_spec(_OUT_BLK, D_HID), _row_spec(_OUT_BLK, D_HID),
              _row_spec(_OUT_BLK, 8), _full_spec(1, D_HID)],
    out_specs=[_row_spec(_OUT_BLK, 16), _row_spec(_OUT_BLK, 16)],
    out_shape=[jax.ShapeDtypeStruct((N_NODES, 16), jnp.float32),
               jax.ShapeDtypeStruct((N_NODES, 16), jnp.float32)],
)


def kernel(x, edge_index, W1, b1, W_mu, b_mu, W_ls, b_ls):
    ei32 = edge_index.astype(jnp.int32)
    src2d = ei32[0].reshape(E_ROWS, CH)
    dst2d = ei32[1].reshape(E_ROWS, CH)
    x_pad = jnp.pad(x, ((0, N_PAD - N_NODES), (0, 0)))
    zeros32 = jnp.zeros((N_PAD, D_HID), jnp.float32)
    zeros8 = jnp.zeros((N_PAD, 8), jnp.float32)
    ones8 = jnp.ones((CH, 8), jnp.float32)
    Wcat = jnp.concatenate([W_mu, W_ls], axis=1)
    bcat = jnp.concatenate([b_mu, b_ls]).reshape(1, D_HID)

    degp = _deg_kernel(dst2d, zeros8, ones8)
    xw1 = _tc_mm(x_pad, W1)
    g1, dinv8 = _tc_scale(degp, xw1)
    acc1p = _spmm_kernel(src2d, dst2d, g1, zeros32)
    g2 = _tc2(acc1p, g1, dinv8, b1.reshape(1, D_HID), Wcat)
    acc2p = _spmm_kernel(src2d, dst2d, g2, zeros32)
    return tuple(_tc3(acc2p, g2, dinv8, bcat))


# speedup vs baseline: 63.1782x; 1.0250x over previous
import functools

import jax
import jax.numpy as jnp
from jax import lax
from jax.experimental import pallas as pl
from jax.experimental.pallas import tpu as pltpu
from jax.experimental.pallas import tpu_sc as plsc

N_NODES = 10000
N_EDGES = 320000
D_IN = 128
D_HID = 32

NC = 2
NS = 16
NW = NC * NS
CH = 256
NB = 8
E_ROWS = N_EDGES // CH
K_MAX = 40
ROW_BLK = 2048
N_PAD = -(-N_NODES // ROW_BLK) * ROW_BLK
NR = N_PAD // NS

_mesh = plsc.VectorSubcoreMesh(
    core_axis_name="c", subcore_axis_name="s", num_cores=NC, num_subcores=NS)
_sc_params = pltpu.CompilerParams(use_tc_tiling_on_sc=False)


def _worker_rows(w):
    wm = jnp.minimum(w, 2)
    base = w * 39 + wm
    nch = 39 + (w < 2).astype(jnp.int32)
    return base, nch


def _load_idx(hbm2d, vref, base, w):
    pltpu.sync_copy(hbm2d.at[pl.ds(base, 39)], vref.at[pl.ds(0, 39)])

    @pl.when(w < 2)
    def _tail():
        pltpu.sync_copy(hbm2d.at[pl.ds(base + 39, 1)], vref.at[pl.ds(39, 1)])


@functools.partial(
    pl.kernel,
    out_type=jax.ShapeDtypeStruct((NC, N_PAD, 8), jnp.float32),
    mesh=_mesh,
    scratch_types=[
        pltpu.VMEM((K_MAX, CH), jnp.int32),
        pltpu.VMEM((CH, 8), jnp.float32),
        pltpu.VMEM_SHARED((N_PAD, 8), jnp.float32),
    ],
    compiler_params=_sc_params,
)
def _deg_kernel(dst_hbm, zeros8_hbm, ones8_hbm, out_hbm, dst_v, ones_v, acc_sh):
    c = lax.axis_index("c")
    s = lax.axis_index("s")
    w = c * NS + s
    r0 = s * NR
    base, nch = _worker_rows(w)
    pltpu.sync_copy(zeros8_hbm.at[pl.ds(r0, NR)], acc_sh.at[pl.ds(r0, NR)])
    _load_idx(dst_hbm, dst_v, base, w)
    pltpu.sync_copy(ones8_hbm, ones_v)
    plsc.subcore_barrier()

    def body(j, carry):
        pltpu.sync_copy(ones_v, acc_sh.at[dst_v.at[j]], add=True)
        return carry

    lax.fori_loop(0, nch, body, 0)
    plsc.subcore_barrier()
    pltpu.sync_copy(acc_sh.at[pl.ds(r0, NR)], out_hbm.at[c, pl.ds(r0, NR)])


@functools.partial(
    pl.kernel,
    out_type=jax.ShapeDtypeStruct((NC, N_PAD, D_HID), jnp.float32),
    mesh=_mesh,
    scratch_types=[
        pltpu.VMEM((K_MAX, CH), jnp.int32),
        pltpu.VMEM((K_MAX, CH), jnp.int32),
        pltpu.VMEM((NB, CH, D_HID), jnp.float32),
        pltpu.VMEM_SHARED((N_PAD, D_HID), jnp.float32),
        pltpu.SemaphoreType.DMA((NB,)),
    ],
    compiler_params=_sc_params,
)
def _spmm_kernel(src_hbm, dst_hbm, g_hbm, zeros_hbm, out_hbm,
                 src_v, dst_v, rows_v, acc_sh, sem_g):
    c = lax.axis_index("c")
    s = lax.axis_index("s")
    w = c * NS + s
    r0 = s * NR
    base, nch = _worker_rows(w)
    pltpu.sync_copy(zeros_hbm.at[pl.ds(r0, NR)], acc_sh.at[pl.ds(r0, NR)])
    _load_idx(src_hbm, src_v, base, w)
    _load_idx(dst_hbm, dst_v, base, w)
    plsc.subcore_barrier()

    for b in range(NB):
        pltpu.async_copy(g_hbm.at[src_v.at[b]], rows_v.at[b], sem_g.at[b])

    def body(j, carry):
        slot = lax.rem(j, NB)
        pltpu.make_async_copy(
            g_hbm.at[src_v.at[j]], rows_v.at[slot], sem_g.at[slot]).wait()
        pltpu.sync_copy(rows_v.at[slot], acc_sh.at[dst_v.at[j]], add=True)

        @pl.when(j + NB < nch)
        def _prefetch():
            pltpu.async_copy(
                g_hbm.at[src_v.at[j + NB]], rows_v.at[slot], sem_g.at[slot])

        return carry

    lax.fori_loop(0, nch, body, 0)
    plsc.subcore_barrier()
    pltpu.sync_copy(acc_sh.at[pl.ds(r0, NR)], out_hbm.at[c, pl.ds(r0, NR)])


def _mm_body(x_ref, w_ref, o_ref):
    o_ref[...] = jnp.dot(x_ref[...], w_ref[...],
                         preferred_element_type=jnp.float32)


def _scale_body(degp_ref, xw_ref, g_ref, dinv_ref):
    d = degp_ref[0] + degp_ref[1]
    dinv = lax.rsqrt(d[:, 0:1] + 1.0)
    g_ref[...] = xw_ref[...] * dinv
    dinv_ref[...] = jnp.broadcast_to(dinv, (ROW_BLK, 8))


def _tc2_body(accp_ref, g1_ref, dinv_ref, b1_ref, w_ref, g2_ref):
    dinv = dinv_ref[:, 0:1]
    h = (accp_ref[0] + accp_ref[1] + g1_ref[...]) * dinv + b1_ref[...]
    h = jnp.maximum(h, 0.0)
    g2_ref[...] = jnp.dot(h, w_ref[...],
                          preferred_element_type=jnp.float32) * dinv


def _tc3_body(accp_ref, g2_ref, dinv_ref, b_ref, mu_ref, ls_ref):
    dinv = dinv_ref[:, 0:1]
    f = (accp_ref[0] + accp_ref[1] + g2_ref[...]) * dinv + b_ref[...]
    mu_ref[...] = f[:, :16]
    ls_ref[...] = f[:, 16:]


def _row_spec(rows, minor):
    return pl.BlockSpec((rows, minor), lambda i: (i, 0))


def _part_spec(rows, minor):
    return pl.BlockSpec((NC, rows, minor), lambda i: (0, i, 0))


def _full_spec(a, b):
    return pl.BlockSpec((a, b), lambda i: (0, 0))


_tc_mm = pl.pallas_call(
    _mm_body,
    grid=(N_PAD // ROW_BLK,),
    in_specs=[_row_spec(ROW_BLK, D_IN), _full_spec(D_IN, D_HID)],
    out_specs=_row_spec(ROW_BLK, D_HID),
    out_shape=jax.ShapeDtypeStruct((N_PAD, D_HID), jnp.float32),
)

_tc_scale = pl.pallas_call(
    _scale_body,
    grid=(N_PAD // ROW_BLK,),
    in_specs=[_part_spec(ROW_BLK, 8), _row_spec(ROW_BLK, D_HID)],
    out_specs=[_row_spec(ROW_BLK, D_HID), _row_spec(ROW_BLK, 8)],
    out_shape=[jax.ShapeDtypeStruct((N_PAD, D_HID), jnp.float32),
               jax.ShapeDtypeStruct((N_PAD, 8), jnp.float32)],
)

_tc2 = pl.pallas_call(
    _tc2_body,
    grid=(N_PAD // ROW_BLK,),
    in_specs=[_part_spec(ROW_BLK, D_HID), _row_spec(ROW_BLK, D_HID),
              _row_spec(ROW_BLK, 8), _full_spec(1, D_HID),
              _full_spec(D_HID, D_HID)],
    out_specs=_row_spec(ROW_BLK, D_HID),
    out_shape=jax.ShapeDtypeStruct((N_PAD, D_HID), jnp.float32),
)

_OUT_BLK = 1000

_tc3 = pl.pallas_call(
    _tc3_body,
    grid=(N_NODES // _OUT_BLK,),
    in_specs=[_part_spec(_OUT_BLK, D_HID), _row_spec(_OUT_BLK, D_HID),
              _row_spec(_OUT_BLK, 8), _full_spec(1, D_HID)],
    out_specs=[_row_spec(_OUT_BLK, 16), _row_spec(_OUT_BLK, 16)],
    out_shape=[jax.ShapeDtypeStruct((N_NODES, 16), jnp.float32),
               jax.ShapeDtypeStruct((N_NODES, 16), jnp.float32)],
)


def kernel(x, edge_index, W1, b1, W_mu, b_mu, W_ls, b_ls):
    ei32 = edge_index.astype(jnp.int32)
    src2d = ei32[0].reshape(E_ROWS, CH)
    dst2d = ei32[1].reshape(E_ROWS, CH)
    x_pad = jnp.pad(x, ((0, N_PAD - N_NODES), (0, 0)))
    zeros32 = jnp.zeros((N_PAD, D_HID), jnp.float32)
    zeros8 = jnp.zeros((N_PAD, 8), jnp.float32)
    ones8 = jnp.ones((CH, 8), jnp.float32)
    Wcat = jnp.concatenate([W_mu, W_ls], axis=1)
    bcat = jnp.concatenate([b_mu, b_ls]).reshape(1, D_HID)

    degp = _deg_kernel(dst2d, zeros8, ones8)
    xw1 = _tc_mm(x_pad, W1)
    g1, dinv8 = _tc_scale(degp, xw1)
    acc1p = _spmm_kernel(src2d, dst2d, g1, zeros32)
    g2 = _tc2(acc1p, g1, dinv8, b1.reshape(1, D_HID), Wcat)
    acc2p = _spmm_kernel(src2d, dst2d, g2, zeros32)
    return tuple(_tc3(acc2p, g2, dinv8, bcat))
